# Initial kernel scaffold; baseline (speedup 1.0000x reference)
import jax
import jax.numpy as jnp
from jax.experimental import pallas as pl


def kernel(nodes, positions, box, edge_shifts, senders, receivers, params):
    raise NotImplementedError("write your pallas kernel here")


# speedup vs baseline: 2.1679x; 2.1679x over previous
import functools

import jax
import jax.numpy as jnp
from jax import lax
from jax.experimental import pallas as pl
from jax.experimental.pallas import tpu as pltpu
from jax.experimental.pallas import tpu_sc as plsc

N_NODES = 10000
N_EDGES = 320000
EDGES_PAD = 327680
IDX_ROWS = EDGES_PAD // 128
NW = 32
CH = IDX_ROWS // NW
EBLK = 2048
EGRID = EDGES_PAD // EBLK
TAB_W = 80
F32 = jnp.float32


def _dot(a, b):
    return jnp.dot(a, b, preferred_element_type=F32)


def _embed_body(nodes_ref, wn_ref, bn_ref, pos_ref, wg_ref, bg_ref,
                tab_ref, glob_ref):
    h = _dot(nodes_ref[...], wn_ref[...]) + bn_ref[...]
    tab_ref[...] = jnp.concatenate([h, pos_ref[...]], axis=1)
    glob_ref[...] = 0.0 * wg_ref[...] + bg_ref[...]


def _tc_embed(nodes, wn, bn, pos_pad, wg, bg):
    return pl.pallas_call(
        _embed_body,
        out_shape=(jax.ShapeDtypeStruct((N_NODES, TAB_W), F32),
                   jax.ShapeDtypeStruct((1, 64), F32)),
    )(nodes, wn, bn, pos_pad, wg, bg)


def _edge_tail(i, eh, sn, rn, gvec_con, w1a, w1b, w1c, w2, b2,
               out_ref, esum_ref):
    y = _dot(eh, w1a) + _dot(sn, w1b) + _dot(rn, w1c) + gvec_con
    h = jnp.maximum(y, 0.0)
    e = _dot(h, w2) + b2
    rows = i * EBLK + lax.broadcasted_iota(jnp.int32, (EBLK, 1), 0)
    e = jnp.where(rows < N_EDGES, e, 0.0)
    out_ref[...] = e

    @pl.when(i == 0)
    def _():
        esum_ref[...] = jnp.zeros_like(esum_ref)
    esum_ref[...] += jnp.sum(e, axis=0, keepdims=True)


def _edge_first_body(gs_ref, gr_ref, sh_ref, boxt_ref, wee_ref, bee_ref,
                     g_ref, w1a_ref, w1b_ref, w1c_ref, w1d_ref, b1_ref,
                     w2_ref, b2_ref, out_ref, esum_ref):
    i = pl.program_id(0)
    gs = gs_ref[...]
    gr = gr_ref[...]
    draw = gr[:, 64:80] - gs[:, 64:80] - sh_ref[...]
    dR = _dot(draw, boxt_ref[...])
    dr2 = jnp.sum(dR * dR, axis=1, keepdims=True) + 1e-12
    dr = jnp.sqrt(dr2)
    k = lax.broadcasted_iota(jnp.int32, (1, 32), 1).astype(F32)
    r0 = 0.05 + k * (3.95 / 29.0)
    d = dr - r0
    rbf = jnp.exp(-(d * d) * 4.0)
    eh = _dot(rbf, wee_ref[...]) + bee_ref[...]
    gcon = _dot(g_ref[...], w1d_ref[...]) + b1_ref[...]
    _edge_tail(i, eh, gs[:, :64], gr[:, :64], gcon,
               w1a_ref[...], w1b_ref[...], w1c_ref[...],
               w2_ref[...], b2_ref[...], out_ref, esum_ref)


def _edge_block_body(eh_ref, gs_ref, gr_ref, g_ref, w1a_ref, w1b_ref,
                     w1c_ref, w1d_ref, b1_ref, w2_ref, b2_ref,
                     out_ref, esum_ref):
    i = pl.program_id(0)
    gcon = _dot(g_ref[...], w1d_ref[...]) + b1_ref[...]
    _edge_tail(i, eh_ref[...], gs_ref[...], gr_ref[...], gcon,
               w1a_ref[...], w1b_ref[...], w1c_ref[...],
               w2_ref[...], b2_ref[...], out_ref, esum_ref)


def _tc_edge_first(gs, gr, shifts_pad, boxt, wee, bee, glob, w1s, b1, w2, b2):
    eb = lambda w: pl.BlockSpec((EBLK, w), lambda i: (i, 0))
    full = lambda a: pl.BlockSpec(a.shape, lambda i: (0,) * a.ndim)
    w1a, w1b, w1c, w1d = w1s
    return pl.pallas_call(
        _edge_first_body,
        grid=(EGRID,),
        in_specs=[eb(TAB_W), eb(TAB_W), eb(16), full(boxt), full(wee),
                  full(bee), full(glob), full(w1a), full(w1b), full(w1c),
                  full(w1d), full(b1), full(w2), full(b2)],
        out_specs=(pl.BlockSpec((EBLK, 64), lambda i: (i, 0)),
                   pl.BlockSpec((1, 64), lambda i: (0, 0))),
        out_shape=(jax.ShapeDtypeStruct((EDGES_PAD, 64), F32),
                   jax.ShapeDtypeStruct((1, 64), F32)),
    )(gs, gr, shifts_pad, boxt, wee, bee, glob, w1a, w1b, w1c, w1d, b1,
      w2, b2)


def _tc_edge_block(eh, gs, gr, glob, w1s, b1, w2, b2):
    eb = lambda w: pl.BlockSpec((EBLK, w), lambda i: (i, 0))
    full = lambda a: pl.BlockSpec(a.shape, lambda i: (0,) * a.ndim)
    w1a, w1b, w1c, w1d = w1s
    return pl.pallas_call(
        _edge_block_body,
        grid=(EGRID,),
        in_specs=[eb(64), eb(64), eb(64), full(glob), full(w1a), full(w1b),
                  full(w1c), full(w1d), full(b1), full(w2), full(b2)],
        out_specs=(pl.BlockSpec((EBLK, 64), lambda i: (i, 0)),
                   pl.BlockSpec((1, 64), lambda i: (0, 0))),
        out_shape=(jax.ShapeDtypeStruct((EDGES_PAD, 64), F32),
                   jax.ShapeDtypeStruct((1, 64), F32)),
    )(eh, gs, gr, glob, w1a, w1b, w1c, w1d, b1, w2, b2)


def _node_body(out_scale, nh_ref, sp0_ref, sp1_ref, rp0_ref, rp1_ref,
               esum_ref, g_ref, v1a_ref, v1b_ref, v1c_ref, v1d_ref, bv1_ref,
               v2_ref, bv2_ref, gn_ref, ge_ref, gg_ref, bg1_ref, g2_ref,
               bg2_ref, nodes_out, glob_out):
    sa = sp0_ref[...] + sp1_ref[...]
    ra = rp0_ref[...] + rp1_ref[...]
    g = g_ref[...]
    y = (_dot(nh_ref[...], v1a_ref[...]) + _dot(sa, v1b_ref[...])
         + _dot(ra, v1c_ref[...]) + _dot(g, v1d_ref[...]) + bv1_ref[...])
    h = jnp.maximum(y, 0.0)
    nn = _dot(h, v2_ref[...]) + bv2_ref[...]
    nodes_out[...] = nn
    nmean = jnp.sum(nn, axis=0, keepdims=True) * (1.0 / N_NODES)
    emean = esum_ref[...] * (1.0 / N_EDGES)
    gy = (_dot(nmean, gn_ref[...]) + _dot(emean, ge_ref[...])
          + _dot(g, gg_ref[...]) + bg1_ref[...])
    gh = jnp.maximum(gy, 0.0)
    go = _dot(gh, g2_ref[...]) + bg2_ref[...]
    glob_out[...] = out_scale * go


def _tc_node(nh, parts, esum, glob, v1s, bv1, v2, bv2, g1s, bg1, g2, bg2,
             out_scale):
    sp0, sp1 = parts[0, 0], parts[0, 1]
    rp0, rp1 = parts[1, 0], parts[1, 1]
    v1a, v1b, v1c, v1d = v1s
    gn, ge, gg = g1s
    return pl.pallas_call(
        functools.partial(_node_body, out_scale),
        out_shape=(jax.ShapeDtypeStruct((N_NODES, 64), F32),
                   jax.ShapeDtypeStruct((1, 64), F32)),
    )(nh, sp0, sp1, rp0, rp1, esum, glob, v1a, v1b, v1c, v1d, bv1, v2,
      bv2, gn, ge, gg, bg1, g2, bg2)


@functools.lru_cache(maxsize=None)
def _make_sc_gather(width):
    mesh = plsc.VectorSubcoreMesh(core_axis_name="c", subcore_axis_name="s")

    def body(tab, is_h, ir_h, os_h, or_h, idxs, idxr, ra, rb, sga, sgb):
        cid = lax.axis_index("c")
        sid = lax.axis_index("s")
        wid = sid * 2 + cid
        pltpu.sync_copy(is_h.at[pl.ds(wid * CH, CH)], idxs)
        pltpu.sync_copy(ir_h.at[pl.ds(wid * CH, CH)], idxr)
        base = wid * CH * 128

        def run(idxv, out):
            def step(c, carry):
                pltpu.async_copy(tab.at[idxv.at[c]], ra, sga).wait()
                pltpu.sync_copy(ra, out.at[pl.ds(base + c * 128, 128)])
                return carry

            lax.fori_loop(0, CH, step, 0)

        run(idxs, os_h)
        run(idxr, or_h)

    return pl.kernel(
        body,
        out_type=(jax.ShapeDtypeStruct((EDGES_PAD, width), F32),
                  jax.ShapeDtypeStruct((EDGES_PAD, width), F32)),
        mesh=mesh,
        scratch_types=[
            pltpu.VMEM((CH, 128), jnp.int32),
            pltpu.VMEM((CH, 128), jnp.int32),
            pltpu.VMEM((128, width), F32),
            pltpu.VMEM((128, width), F32),
            pltpu.SemaphoreType.DMA,
            pltpu.SemaphoreType.DMA,
        ],
        compiler_params=pltpu.CompilerParams(use_tc_tiling_on_sc=False),
    )


def _sc_gather_pair(table, idx2_s, idx2_r):
    return _make_sc_gather(table.shape[1])(table, idx2_s, idx2_r)


@functools.lru_cache(maxsize=None)
def _make_sc_scatter():
    mesh = plsc.VectorSubcoreMesh(core_axis_name="c", subcore_axis_name="s")
    stripe = N_NODES // 16

    def body(edges_h, is_h, ir_h, zeros_h, out_h, acc_s, acc_r, idxs, idxr,
             rowbuf):
        cid = lax.axis_index("c")
        sid = lax.axis_index("s")
        wid = sid * 2 + cid
        pltpu.sync_copy(zeros_h.at[pl.ds(sid * stripe, stripe)],
                        acc_s.at[pl.ds(sid * stripe, stripe)])
        pltpu.sync_copy(zeros_h.at[pl.ds(sid * stripe, stripe)],
                        acc_r.at[pl.ds(sid * stripe, stripe)])
        pltpu.sync_copy(is_h.at[pl.ds(wid * CH, CH)], idxs)
        pltpu.sync_copy(ir_h.at[pl.ds(wid * CH, CH)], idxr)
        plsc.subcore_barrier()
        base = wid * CH * 128

        def step(c, carry):
            pltpu.sync_copy(edges_h.at[pl.ds(base + c * 128, 128)], rowbuf)
            pltpu.sync_copy(rowbuf, acc_s.at[idxs.at[c]], add=True)
            pltpu.sync_copy(rowbuf, acc_r.at[idxr.at[c]], add=True)
            return carry

        lax.fori_loop(0, CH, step, 0)
        plsc.subcore_barrier()
        pltpu.sync_copy(acc_s.at[pl.ds(sid * stripe, stripe)],
                        out_h.at[0, cid, pl.ds(sid * stripe, stripe)])
        pltpu.sync_copy(acc_r.at[pl.ds(sid * stripe, stripe)],
                        out_h.at[1, cid, pl.ds(sid * stripe, stripe)])

    return pl.kernel(
        body,
        out_type=jax.ShapeDtypeStruct((2, 2, N_NODES, 64), F32),
        mesh=mesh,
        scratch_types=[
            pltpu.VMEM_SHARED((N_NODES, 64), F32),
            pltpu.VMEM_SHARED((N_NODES, 64), F32),
            pltpu.VMEM((CH, 128), jnp.int32),
            pltpu.VMEM((CH, 128), jnp.int32),
            pltpu.VMEM((128, 64), F32),
        ],
        compiler_params=pltpu.CompilerParams(use_tc_tiling_on_sc=False),
    )


def _sc_scatter_pair(edges, idx2_s, idx2_r, zeros_tab):
    return _make_sc_scatter()(edges, idx2_s, idx2_r, zeros_tab)


def _split_w1(w):
    return (w[0:64], w[64:128], w[128:192], w[192:256])


def _split_g1(w):
    return (w[0:64], w[64:128], w[128:192])


def _b2d(b):
    return b.reshape(1, -1)


def kernel(nodes, positions, box, edge_shifts, senders, receivers, params):
    p = params
    pad = EDGES_PAD - N_EDGES
    s32 = senders.astype(jnp.int32)
    r32 = receivers.astype(jnp.int32)
    idx_s = jnp.concatenate([s32, jnp.zeros((pad,), jnp.int32)]).reshape(
        IDX_ROWS, 128)
    idx_r = jnp.concatenate([r32, jnp.zeros((pad,), jnp.int32)]).reshape(
        IDX_ROWS, 128)
    shifts_pad = jnp.pad(edge_shifts, ((0, pad), (0, 13)))
    pos_pad = jnp.pad(positions, ((0, 0), (0, 13)))
    boxt = jnp.pad(box[0].T, ((0, 13), (0, 13)))
    zeros_tab = jnp.zeros((N_NODES, 64), F32)
    wee = jnp.pad(p['edge_emb']['W'], ((0, 2), (0, 0)))

    tab0, glob0 = _tc_embed(nodes, p['node_emb']['W'],
                            _b2d(p['node_emb']['b']), pos_pad,
                            p['glob_emb']['W'], _b2d(p['glob_emb']['b']))

    gs, gr = _sc_gather_pair(tab0, idx_s, idx_r)

    sp = p['steps'][0]
    edges_h, esum = _tc_edge_first(
        gs, gr, shifts_pad, boxt, wee, _b2d(p['edge_emb']['b']), glob0,
        _split_w1(sp['edge'][0]['W']), _b2d(sp['edge'][0]['b']),
        sp['edge'][1]['W'], _b2d(sp['edge'][1]['b']))

    nodes_h = tab0[:, :64]
    glob_h = glob0
    n_steps = len(p['steps'])
    for step_i in range(n_steps + 1):
        sp = p['steps'][step_i] if step_i < n_steps else p['readout']
        is_last = step_i == n_steps
        parts = _sc_scatter_pair(edges_h, idx_s, idx_r, zeros_tab)
        g2 = sp['global'][1]['W']
        bg2 = _b2d(sp['global'][1]['b'])
        if is_last:
            g2 = jnp.pad(g2, ((0, 0), (0, 64 - g2.shape[1])))
            bg2 = jnp.pad(bg2, ((0, 0), (0, 64 - bg2.shape[1])))
        nodes_h, glob_h = _tc_node(
            nodes_h, parts, esum, glob_h,
            _split_w1(sp['node'][0]['W']), _b2d(sp['node'][0]['b']),
            sp['node'][1]['W'], _b2d(sp['node'][1]['b']),
            _split_g1(sp['global'][0]['W']), _b2d(sp['global'][0]['b']),
            g2, bg2,
            out_scale=float(N_NODES) if is_last else 1.0)
        if is_last:
            break
        nsp = p['steps'][step_i + 1] if step_i + 1 < n_steps else p['readout']
        gs, gr = _sc_gather_pair(nodes_h, idx_s, idx_r)
        edges_h, esum = _tc_edge_block(
            edges_h, gs, gr, glob_h,
            _split_w1(nsp['edge'][0]['W']), _b2d(nsp['edge'][0]['b']),
            nsp['edge'][1]['W'], _b2d(nsp['edge'][1]['b']))

    return glob_h[:, :1]


# speedup vs baseline: 2.3420x; 1.0803x over previous
import functools

import jax
import jax.numpy as jnp
from jax import lax
from jax.experimental import pallas as pl
from jax.experimental.pallas import tpu as pltpu
from jax.experimental.pallas import tpu_sc as plsc

N_NODES = 10000
N_EDGES = 320000
EDGES_PAD = 327680
IDX_ROWS = EDGES_PAD // 128
NW = 32
CH = IDX_ROWS // NW
EBLK = 2048
EGRID = EDGES_PAD // EBLK
TAB_W = 80
F32 = jnp.float32


def _dot(a, b):
    return jnp.dot(a, b, preferred_element_type=F32)


def _embed_body(nodes_ref, wn_ref, bn_ref, pos_ref, wg_ref, bg_ref,
                tab_ref, glob_ref):
    h = _dot(nodes_ref[...], wn_ref[...]) + bn_ref[...]
    tab_ref[...] = jnp.concatenate([h, pos_ref[...]], axis=1)
    glob_ref[...] = 0.0 * wg_ref[...] + bg_ref[...]


def _tc_embed(nodes, wn, bn, pos_pad, wg, bg):
    return pl.pallas_call(
        _embed_body,
        out_shape=(jax.ShapeDtypeStruct((N_NODES, TAB_W), F32),
                   jax.ShapeDtypeStruct((1, 64), F32)),
    )(nodes, wn, bn, pos_pad, wg, bg)


def _edge_tail(i, eh, sn, rn, gvec_con, w1a, w1b, w1c, w2, b2,
               out_ref, esum_ref):
    y = _dot(eh, w1a) + _dot(sn, w1b) + _dot(rn, w1c) + gvec_con
    h = jnp.maximum(y, 0.0)
    e = _dot(h, w2) + b2
    rows = i * EBLK + lax.broadcasted_iota(jnp.int32, (EBLK, 1), 0)
    e = jnp.where(rows < N_EDGES, e, 0.0)
    out_ref[...] = e

    @pl.when(i == 0)
    def _():
        esum_ref[...] = jnp.zeros_like(esum_ref)
    esum_ref[...] += jnp.sum(e, axis=0, keepdims=True)


def _edge_first_body(gs_ref, gr_ref, sh_ref, boxt_ref, wee_ref, bee_ref,
                     g_ref, w1a_ref, w1b_ref, w1c_ref, w1d_ref, b1_ref,
                     w2_ref, b2_ref, out_ref, esum_ref):
    i = pl.program_id(0)
    gs = gs_ref[...]
    gr = gr_ref[...]
    draw = gr[:, 64:80] - gs[:, 64:80] - sh_ref[...]
    dR = _dot(draw, boxt_ref[...])
    dr2 = jnp.sum(dR * dR, axis=1, keepdims=True) + 1e-12
    dr = jnp.sqrt(dr2)
    k = lax.broadcasted_iota(jnp.int32, (1, 32), 1).astype(F32)
    r0 = 0.05 + k * (3.95 / 29.0)
    d = dr - r0
    rbf = jnp.exp(-(d * d) * 4.0)
    eh = _dot(rbf, wee_ref[...]) + bee_ref[...]
    gcon = _dot(g_ref[...], w1d_ref[...]) + b1_ref[...]
    _edge_tail(i, eh, gs[:, :64], gr[:, :64], gcon,
               w1a_ref[...], w1b_ref[...], w1c_ref[...],
               w2_ref[...], b2_ref[...], out_ref, esum_ref)


def _edge_block_body(eh_ref, gs_ref, gr_ref, g_ref, w1a_ref, w1b_ref,
                     w1c_ref, w1d_ref, b1_ref, w2_ref, b2_ref,
                     out_ref, esum_ref):
    i = pl.program_id(0)
    gcon = _dot(g_ref[...], w1d_ref[...]) + b1_ref[...]
    _edge_tail(i, eh_ref[...], gs_ref[...], gr_ref[...], gcon,
               w1a_ref[...], w1b_ref[...], w1c_ref[...],
               w2_ref[...], b2_ref[...], out_ref, esum_ref)


def _tc_edge_first(gs, gr, shifts_pad, boxt, wee, bee, glob, w1s, b1, w2, b2):
    eb = lambda w: pl.BlockSpec((EBLK, w), lambda i: (i, 0))
    full = lambda a: pl.BlockSpec(a.shape, lambda i: (0,) * a.ndim)
    w1a, w1b, w1c, w1d = w1s
    return pl.pallas_call(
        _edge_first_body,
        grid=(EGRID,),
        in_specs=[eb(TAB_W), eb(TAB_W), eb(16), full(boxt), full(wee),
                  full(bee), full(glob), full(w1a), full(w1b), full(w1c),
                  full(w1d), full(b1), full(w2), full(b2)],
        out_specs=(pl.BlockSpec((EBLK, 64), lambda i: (i, 0)),
                   pl.BlockSpec((1, 64), lambda i: (0, 0))),
        out_shape=(jax.ShapeDtypeStruct((EDGES_PAD, 64), F32),
                   jax.ShapeDtypeStruct((1, 64), F32)),
    )(gs, gr, shifts_pad, boxt, wee, bee, glob, w1a, w1b, w1c, w1d, b1,
      w2, b2)


def _tc_edge_block(eh, gs, gr, glob, w1s, b1, w2, b2):
    eb = lambda w: pl.BlockSpec((EBLK, w), lambda i: (i, 0))
    full = lambda a: pl.BlockSpec(a.shape, lambda i: (0,) * a.ndim)
    w1a, w1b, w1c, w1d = w1s
    return pl.pallas_call(
        _edge_block_body,
        grid=(EGRID,),
        in_specs=[eb(64), eb(64), eb(64), full(glob), full(w1a), full(w1b),
                  full(w1c), full(w1d), full(b1), full(w2), full(b2)],
        out_specs=(pl.BlockSpec((EBLK, 64), lambda i: (i, 0)),
                   pl.BlockSpec((1, 64), lambda i: (0, 0))),
        out_shape=(jax.ShapeDtypeStruct((EDGES_PAD, 64), F32),
                   jax.ShapeDtypeStruct((1, 64), F32)),
    )(eh, gs, gr, glob, w1a, w1b, w1c, w1d, b1, w2, b2)


def _node_body(out_scale, nh_ref, sp0_ref, sp1_ref, rp0_ref, rp1_ref,
               esum_ref, g_ref, v1a_ref, v1b_ref, v1c_ref, v1d_ref, bv1_ref,
               v2_ref, bv2_ref, gn_ref, ge_ref, gg_ref, bg1_ref, g2_ref,
               bg2_ref, nodes_out, glob_out):
    sa = sp0_ref[...] + sp1_ref[...]
    ra = rp0_ref[...] + rp1_ref[...]
    g = g_ref[...]
    y = (_dot(nh_ref[...], v1a_ref[...]) + _dot(sa, v1b_ref[...])
         + _dot(ra, v1c_ref[...]) + _dot(g, v1d_ref[...]) + bv1_ref[...])
    h = jnp.maximum(y, 0.0)
    nn = _dot(h, v2_ref[...]) + bv2_ref[...]
    nodes_out[...] = nn
    nmean = jnp.sum(nn, axis=0, keepdims=True) * (1.0 / N_NODES)
    emean = esum_ref[...] * (1.0 / N_EDGES)
    gy = (_dot(nmean, gn_ref[...]) + _dot(emean, ge_ref[...])
          + _dot(g, gg_ref[...]) + bg1_ref[...])
    gh = jnp.maximum(gy, 0.0)
    go = _dot(gh, g2_ref[...]) + bg2_ref[...]
    glob_out[...] = out_scale * go


def _tc_node(nh, parts, esum, glob, v1s, bv1, v2, bv2, g1s, bg1, g2, bg2,
             out_scale):
    sp0, sp1 = parts[0, 0], parts[0, 1]
    rp0, rp1 = parts[1, 0], parts[1, 1]
    v1a, v1b, v1c, v1d = v1s
    gn, ge, gg = g1s
    return pl.pallas_call(
        functools.partial(_node_body, out_scale),
        out_shape=(jax.ShapeDtypeStruct((N_NODES, 64), F32),
                   jax.ShapeDtypeStruct((1, 64), F32)),
    )(nh, sp0, sp1, rp0, rp1, esum, glob, v1a, v1b, v1c, v1d, bv1, v2,
      bv2, gn, ge, gg, bg1, g2, bg2)


@functools.lru_cache(maxsize=None)
def _make_sc_gather(width):
    mesh = plsc.VectorSubcoreMesh(core_axis_name="c", subcore_axis_name="s")

    def body(tab, is_h, ir_h, os_h, or_h, idxs, idxr, ra, rb, sga, sgb):
        cid = lax.axis_index("c")
        sid = lax.axis_index("s")
        wid = sid * 2 + cid
        pltpu.sync_copy(is_h.at[pl.ds(wid * CH, CH)], idxs)
        pltpu.sync_copy(ir_h.at[pl.ds(wid * CH, CH)], idxr)
        base = wid * CH * 128

        def run(idxv, out):
            def fire(buf, sem, cbase):
                for b in range(4):
                    pltpu.async_copy(tab.at[idxv.at[cbase + b]],
                                     buf.at[pl.ds(b * 128, 128)], sem)

            def drain_store(buf, sem, cbase):
                for b in range(4):
                    pltpu.make_async_copy(tab.at[idxv.at[0]],
                                          buf.at[pl.ds(b * 128, 128)],
                                          sem).wait()
                pltpu.sync_copy(buf, out.at[pl.ds(base + cbase * 128, 512)])

            fire(ra, sga, 0)

            def step(j, carry):
                c = j * 8
                fire(rb, sgb, c + 4)
                drain_store(ra, sga, c)

                @pl.when(j < CH // 8 - 1)
                def _():
                    fire(ra, sga, c + 8)
                drain_store(rb, sgb, c + 4)
                return carry

            lax.fori_loop(0, CH // 8, step, 0)

        run(idxs, os_h)
        run(idxr, or_h)

    return pl.kernel(
        body,
        out_type=(jax.ShapeDtypeStruct((EDGES_PAD, width), F32),
                  jax.ShapeDtypeStruct((EDGES_PAD, width), F32)),
        mesh=mesh,
        scratch_types=[
            pltpu.VMEM((CH, 128), jnp.int32),
            pltpu.VMEM((CH, 128), jnp.int32),
            pltpu.VMEM((512, width), F32),
            pltpu.VMEM((512, width), F32),
            pltpu.SemaphoreType.DMA,
            pltpu.SemaphoreType.DMA,
        ],
        compiler_params=pltpu.CompilerParams(use_tc_tiling_on_sc=False),
    )


def _sc_gather_pair(table, idx2_s, idx2_r):
    return _make_sc_gather(table.shape[1])(table, idx2_s, idx2_r)


@functools.lru_cache(maxsize=None)
def _make_sc_scatter():
    mesh = plsc.VectorSubcoreMesh(core_axis_name="c", subcore_axis_name="s")
    stripe = N_NODES // 16

    def body(edges_h, is_h, ir_h, zeros_h, out_h, acc_s, acc_r, idxs, idxr,
             rowbuf):
        cid = lax.axis_index("c")
        sid = lax.axis_index("s")
        wid = sid * 2 + cid
        pltpu.sync_copy(zeros_h.at[pl.ds(sid * stripe, stripe)],
                        acc_s.at[pl.ds(sid * stripe, stripe)])
        pltpu.sync_copy(zeros_h.at[pl.ds(sid * stripe, stripe)],
                        acc_r.at[pl.ds(sid * stripe, stripe)])
        pltpu.sync_copy(is_h.at[pl.ds(wid * CH, CH)], idxs)
        pltpu.sync_copy(ir_h.at[pl.ds(wid * CH, CH)], idxr)
        plsc.subcore_barrier()
        base = wid * CH * 128

        def step(c, carry):
            pltpu.sync_copy(edges_h.at[pl.ds(base + c * 128, 128)], rowbuf)
            pltpu.sync_copy(rowbuf, acc_s.at[idxs.at[c]], add=True)
            pltpu.sync_copy(rowbuf, acc_r.at[idxr.at[c]], add=True)
            return carry

        lax.fori_loop(0, CH, step, 0)
        plsc.subcore_barrier()
        pltpu.sync_copy(acc_s.at[pl.ds(sid * stripe, stripe)],
                        out_h.at[0, cid, pl.ds(sid * stripe, stripe)])
        pltpu.sync_copy(acc_r.at[pl.ds(sid * stripe, stripe)],
                        out_h.at[1, cid, pl.ds(sid * stripe, stripe)])

    return pl.kernel(
        body,
        out_type=jax.ShapeDtypeStruct((2, 2, N_NODES, 64), F32),
        mesh=mesh,
        scratch_types=[
            pltpu.VMEM_SHARED((N_NODES, 64), F32),
            pltpu.VMEM_SHARED((N_NODES, 64), F32),
            pltpu.VMEM((CH, 128), jnp.int32),
            pltpu.VMEM((CH, 128), jnp.int32),
            pltpu.VMEM((128, 64), F32),
        ],
        compiler_params=pltpu.CompilerParams(use_tc_tiling_on_sc=False),
    )


def _sc_scatter_pair(edges, idx2_s, idx2_r, zeros_tab):
    return _make_sc_scatter()(edges, idx2_s, idx2_r, zeros_tab)


def _split_w1(w):
    return (w[0:64], w[64:128], w[128:192], w[192:256])


def _split_g1(w):
    return (w[0:64], w[64:128], w[128:192])


def _b2d(b):
    return b.reshape(1, -1)


def kernel(nodes, positions, box, edge_shifts, senders, receivers, params):
    p = params
    pad = EDGES_PAD - N_EDGES
    s32 = senders.astype(jnp.int32)
    r32 = receivers.astype(jnp.int32)
    idx_s = jnp.concatenate([s32, jnp.zeros((pad,), jnp.int32)]).reshape(
        IDX_ROWS, 128)
    idx_r = jnp.concatenate([r32, jnp.zeros((pad,), jnp.int32)]).reshape(
        IDX_ROWS, 128)
    shifts_pad = jnp.pad(edge_shifts, ((0, pad), (0, 13)))
    pos_pad = jnp.pad(positions, ((0, 0), (0, 13)))
    boxt = jnp.pad(box[0].T, ((0, 13), (0, 13)))
    zeros_tab = jnp.zeros((N_NODES, 64), F32)
    wee = jnp.pad(p['edge_emb']['W'], ((0, 2), (0, 0)))

    tab0, glob0 = _tc_embed(nodes, p['node_emb']['W'],
                            _b2d(p['node_emb']['b']), pos_pad,
                            p['glob_emb']['W'], _b2d(p['glob_emb']['b']))

    gs, gr = _sc_gather_pair(tab0, idx_s, idx_r)

    sp = p['steps'][0]
    edges_h, esum = _tc_edge_first(
        gs, gr, shifts_pad, boxt, wee, _b2d(p['edge_emb']['b']), glob0,
        _split_w1(sp['edge'][0]['W']), _b2d(sp['edge'][0]['b']),
        sp['edge'][1]['W'], _b2d(sp['edge'][1]['b']))

    nodes_h = tab0[:, :64]
    glob_h = glob0
    n_steps = len(p['steps'])
    for step_i in range(n_steps + 1):
        sp = p['steps'][step_i] if step_i < n_steps else p['readout']
        is_last = step_i == n_steps
        parts = _sc_scatter_pair(edges_h, idx_s, idx_r, zeros_tab)
        g2 = sp['global'][1]['W']
        bg2 = _b2d(sp['global'][1]['b'])
        if is_last:
            g2 = jnp.pad(g2, ((0, 0), (0, 64 - g2.shape[1])))
            bg2 = jnp.pad(bg2, ((0, 0), (0, 64 - bg2.shape[1])))
        nodes_h, glob_h = _tc_node(
            nodes_h, parts, esum, glob_h,
            _split_w1(sp['node'][0]['W']), _b2d(sp['node'][0]['b']),
            sp['node'][1]['W'], _b2d(sp['node'][1]['b']),
            _split_g1(sp['global'][0]['W']), _b2d(sp['global'][0]['b']),
            g2, bg2,
            out_scale=float(N_NODES) if is_last else 1.0)
        if is_last:
            break
        nsp = p['steps'][step_i + 1] if step_i + 1 < n_steps else p['readout']
        gs, gr = _sc_gather_pair(nodes_h, idx_s, idx_r)
        edges_h, esum = _tc_edge_block(
            edges_h, gs, gr, glob_h,
            _split_w1(nsp['edge'][0]['W']), _b2d(nsp['edge'][0]['b']),
            nsp['edge'][1]['W'], _b2d(nsp['edge'][1]['b']))

    return glob_h[:, :1]


# speedup vs baseline: 3.4347x; 1.4665x over previous
import functools

import jax
import jax.numpy as jnp
from jax import lax
from jax.experimental import pallas as pl
from jax.experimental.pallas import tpu as pltpu
from jax.experimental.pallas import tpu_sc as plsc

N_NODES = 10000
N_EDGES = 320000
EDGES_PAD = 327680
IDX_ROWS = EDGES_PAD // 128
NW = 32
CH = IDX_ROWS // NW
EBLK = 2048
EGRID = EDGES_PAD // EBLK
TAB_W = 80
F32 = jnp.float32


def _dot(a, b):
    return jnp.dot(a, b, preferred_element_type=F32)


def _embed_body(nodes_ref, wn_ref, bn_ref, pos_ref, wg_ref, bg_ref,
                tab_ref, glob_ref):
    h = _dot(nodes_ref[...], wn_ref[...]) + bn_ref[...]
    tab_ref[...] = jnp.concatenate([h, pos_ref[...]], axis=1)
    glob_ref[...] = 0.0 * wg_ref[...] + bg_ref[...]


def _tc_embed(nodes, wn, bn, pos_pad, wg, bg):
    return pl.pallas_call(
        _embed_body,
        out_shape=(jax.ShapeDtypeStruct((N_NODES, TAB_W), F32),
                   jax.ShapeDtypeStruct((1, 64), F32)),
    )(nodes, wn, bn, pos_pad, wg, bg)


def _edge_tail(i, eh, sn, rn, gvec_con, w1a, w1b, w1c, w2, b2,
               out_ref, esum_ref):
    y = _dot(eh, w1a) + _dot(sn, w1b) + _dot(rn, w1c) + gvec_con
    h = jnp.maximum(y, 0.0)
    e = _dot(h, w2) + b2
    rows = i * EBLK + lax.broadcasted_iota(jnp.int32, (EBLK, 1), 0)
    e = jnp.where(rows < N_EDGES, e, 0.0)
    out_ref[...] = e

    @pl.when(i == 0)
    def _():
        esum_ref[...] = jnp.zeros_like(esum_ref)
    esum_ref[...] += jnp.sum(e, axis=0, keepdims=True)


def _edge_first_body(gs_ref, gr_ref, sh_ref, boxt_ref, wee_ref, bee_ref,
                     g_ref, w1a_ref, w1b_ref, w1c_ref, w1d_ref, b1_ref,
                     w2_ref, b2_ref, out_ref, esum_ref):
    i = pl.program_id(0)
    gs = gs_ref[...]
    gr = gr_ref[...]
    draw = gr[:, 64:80] - gs[:, 64:80] - sh_ref[...]
    dR = _dot(draw, boxt_ref[...])
    dr2 = jnp.sum(dR * dR, axis=1, keepdims=True) + 1e-12
    dr = jnp.sqrt(dr2)
    k = lax.broadcasted_iota(jnp.int32, (1, 32), 1).astype(F32)
    r0 = 0.05 + k * (3.95 / 29.0)
    d = dr - r0
    rbf = jnp.exp(-(d * d) * 4.0)
    eh = _dot(rbf, wee_ref[...]) + bee_ref[...]
    gcon = _dot(g_ref[...], w1d_ref[...]) + b1_ref[...]
    _edge_tail(i, eh, gs[:, :64], gr[:, :64], gcon,
               w1a_ref[...], w1b_ref[...], w1c_ref[...],
               w2_ref[...], b2_ref[...], out_ref, esum_ref)


def _edge_block_body(eh_ref, gs_ref, gr_ref, g_ref, w1a_ref, w1b_ref,
                     w1c_ref, w1d_ref, b1_ref, w2_ref, b2_ref,
                     out_ref, esum_ref):
    i = pl.program_id(0)
    gcon = _dot(g_ref[...], w1d_ref[...]) + b1_ref[...]
    _edge_tail(i, eh_ref[...], gs_ref[...], gr_ref[...], gcon,
               w1a_ref[...], w1b_ref[...], w1c_ref[...],
               w2_ref[...], b2_ref[...], out_ref, esum_ref)


def _tc_edge_first(gs, gr, shifts_pad, boxt, wee, bee, glob, w1s, b1, w2, b2):
    eb = lambda w: pl.BlockSpec((EBLK, w), lambda i: (i, 0))
    full = lambda a: pl.BlockSpec(a.shape, lambda i: (0,) * a.ndim)
    w1a, w1b, w1c, w1d = w1s
    return pl.pallas_call(
        _edge_first_body,
        grid=(EGRID,),
        in_specs=[eb(TAB_W), eb(TAB_W), eb(16), full(boxt), full(wee),
                  full(bee), full(glob), full(w1a), full(w1b), full(w1c),
                  full(w1d), full(b1), full(w2), full(b2)],
        out_specs=(pl.BlockSpec((EBLK, 64), lambda i: (i, 0)),
                   pl.BlockSpec((1, 64), lambda i: (0, 0))),
        out_shape=(jax.ShapeDtypeStruct((EDGES_PAD, 64), F32),
                   jax.ShapeDtypeStruct((1, 64), F32)),
    )(gs, gr, shifts_pad, boxt, wee, bee, glob, w1a, w1b, w1c, w1d, b1,
      w2, b2)


def _tc_edge_block(eh, gs, gr, glob, w1s, b1, w2, b2):
    eb = lambda w: pl.BlockSpec((EBLK, w), lambda i: (i, 0))
    full = lambda a: pl.BlockSpec(a.shape, lambda i: (0,) * a.ndim)
    w1a, w1b, w1c, w1d = w1s
    return pl.pallas_call(
        _edge_block_body,
        grid=(EGRID,),
        in_specs=[eb(64), eb(64), eb(64), full(glob), full(w1a), full(w1b),
                  full(w1c), full(w1d), full(b1), full(w2), full(b2)],
        out_specs=(pl.BlockSpec((EBLK, 64), lambda i: (i, 0)),
                   pl.BlockSpec((1, 64), lambda i: (0, 0))),
        out_shape=(jax.ShapeDtypeStruct((EDGES_PAD, 64), F32),
                   jax.ShapeDtypeStruct((1, 64), F32)),
    )(eh, gs, gr, glob, w1a, w1b, w1c, w1d, b1, w2, b2)


def _node_body(out_scale, nh_ref, sp0_ref, sp1_ref, rp0_ref, rp1_ref,
               esum_ref, g_ref, v1a_ref, v1b_ref, v1c_ref, v1d_ref, bv1_ref,
               v2_ref, bv2_ref, gn_ref, ge_ref, gg_ref, bg1_ref, g2_ref,
               bg2_ref, nodes_out, glob_out):
    sa = sp0_ref[...] + sp1_ref[...]
    ra = rp0_ref[...] + rp1_ref[...]
    g = g_ref[...]
    y = (_dot(nh_ref[...], v1a_ref[...]) + _dot(sa, v1b_ref[...])
         + _dot(ra, v1c_ref[...]) + _dot(g, v1d_ref[...]) + bv1_ref[...])
    h = jnp.maximum(y, 0.0)
    nn = _dot(h, v2_ref[...]) + bv2_ref[...]
    nodes_out[...] = nn
    nmean = jnp.sum(nn, axis=0, keepdims=True) * (1.0 / N_NODES)
    emean = esum_ref[...] * (1.0 / N_EDGES)
    gy = (_dot(nmean, gn_ref[...]) + _dot(emean, ge_ref[...])
          + _dot(g, gg_ref[...]) + bg1_ref[...])
    gh = jnp.maximum(gy, 0.0)
    go = _dot(gh, g2_ref[...]) + bg2_ref[...]
    glob_out[...] = out_scale * go


def _tc_node(nh, parts, esum, glob, v1s, bv1, v2, bv2, g1s, bg1, g2, bg2,
             out_scale):
    sp0, sp1 = parts[0, 0], parts[0, 1]
    rp0, rp1 = parts[1, 0], parts[1, 1]
    v1a, v1b, v1c, v1d = v1s
    gn, ge, gg = g1s
    return pl.pallas_call(
        functools.partial(_node_body, out_scale),
        out_shape=(jax.ShapeDtypeStruct((N_NODES, 64), F32),
                   jax.ShapeDtypeStruct((1, 64), F32)),
    )(nh, sp0, sp1, rp0, rp1, esum, glob, v1a, v1b, v1c, v1d, bv1, v2,
      bv2, gn, ge, gg, bg1, g2, bg2)


@functools.lru_cache(maxsize=None)
def _make_sc_gather(width):
    mesh = plsc.VectorSubcoreMesh(core_axis_name="c", subcore_axis_name="s")

    stripe = N_NODES // 16

    def body(tab, is_h, ir_h, os_h, or_h, tab_sp, idxs, idxr, ra, rb, sga,
             sgb):
        cid = lax.axis_index("c")
        sid = lax.axis_index("s")
        wid = sid * 2 + cid
        pltpu.sync_copy(tab.at[pl.ds(sid * stripe, stripe)],
                        tab_sp.at[pl.ds(sid * stripe, stripe)])
        pltpu.sync_copy(is_h.at[pl.ds(wid * CH, CH)], idxs)
        pltpu.sync_copy(ir_h.at[pl.ds(wid * CH, CH)], idxr)
        plsc.subcore_barrier()
        base = wid * CH * 128

        def run(idxv, out):
            def fire(buf, sem, cbase):
                for b in range(2):
                    pltpu.async_copy(tab_sp.at[idxv.at[cbase + b]],
                                     buf.at[pl.ds(b * 128, 128)], sem)

            def drain_store(buf, sem, cbase):
                for b in range(2):
                    pltpu.make_async_copy(tab_sp.at[idxv.at[0]],
                                          buf.at[pl.ds(b * 128, 128)],
                                          sem).wait()
                pltpu.sync_copy(buf, out.at[pl.ds(base + cbase * 128, 256)])

            fire(ra, sga, 0)

            def step(j, carry):
                c = j * 4
                fire(rb, sgb, c + 2)
                drain_store(ra, sga, c)

                @pl.when(j < CH // 4 - 1)
                def _():
                    fire(ra, sga, c + 4)
                drain_store(rb, sgb, c + 2)
                return carry

            lax.fori_loop(0, CH // 4, step, 0)

        run(idxs, os_h)
        run(idxr, or_h)

    return pl.kernel(
        body,
        out_type=(jax.ShapeDtypeStruct((EDGES_PAD, width), F32),
                  jax.ShapeDtypeStruct((EDGES_PAD, width), F32)),
        mesh=mesh,
        scratch_types=[
            pltpu.VMEM_SHARED((N_NODES, width), F32),
            pltpu.VMEM((CH, 128), jnp.int32),
            pltpu.VMEM((CH, 128), jnp.int32),
            pltpu.VMEM((256, width), F32),
            pltpu.VMEM((256, width), F32),
            pltpu.SemaphoreType.DMA,
            pltpu.SemaphoreType.DMA,
        ],
        compiler_params=pltpu.CompilerParams(use_tc_tiling_on_sc=False),
    )


def _sc_gather_pair(table, idx2_s, idx2_r):
    return _make_sc_gather(table.shape[1])(table, idx2_s, idx2_r)


@functools.lru_cache(maxsize=None)
def _make_sc_scatter():
    mesh = plsc.VectorSubcoreMesh(core_axis_name="c", subcore_axis_name="s")
    stripe = N_NODES // 16

    def body(edges_h, is_h, ir_h, zeros_h, out_h, acc_s, acc_r, idxs, idxr,
             rowbuf):
        cid = lax.axis_index("c")
        sid = lax.axis_index("s")
        wid = sid * 2 + cid
        pltpu.sync_copy(zeros_h.at[pl.ds(sid * stripe, stripe)],
                        acc_s.at[pl.ds(sid * stripe, stripe)])
        pltpu.sync_copy(zeros_h.at[pl.ds(sid * stripe, stripe)],
                        acc_r.at[pl.ds(sid * stripe, stripe)])
        pltpu.sync_copy(is_h.at[pl.ds(wid * CH, CH)], idxs)
        pltpu.sync_copy(ir_h.at[pl.ds(wid * CH, CH)], idxr)
        plsc.subcore_barrier()
        base = wid * CH * 128

        def step(c, carry):
            pltpu.sync_copy(edges_h.at[pl.ds(base + c * 128, 128)], rowbuf)
            pltpu.sync_copy(rowbuf, acc_s.at[idxs.at[c]], add=True)
            pltpu.sync_copy(rowbuf, acc_r.at[idxr.at[c]], add=True)
            return carry

        lax.fori_loop(0, CH, step, 0)
        plsc.subcore_barrier()
        pltpu.sync_copy(acc_s.at[pl.ds(sid * stripe, stripe)],
                        out_h.at[0, cid, pl.ds(sid * stripe, stripe)])
        pltpu.sync_copy(acc_r.at[pl.ds(sid * stripe, stripe)],
                        out_h.at[1, cid, pl.ds(sid * stripe, stripe)])

    return pl.kernel(
        body,
        out_type=jax.ShapeDtypeStruct((2, 2, N_NODES, 64), F32),
        mesh=mesh,
        scratch_types=[
            pltpu.VMEM_SHARED((N_NODES, 64), F32),
            pltpu.VMEM_SHARED((N_NODES, 64), F32),
            pltpu.VMEM((CH, 128), jnp.int32),
            pltpu.VMEM((CH, 128), jnp.int32),
            pltpu.VMEM((128, 64), F32),
        ],
        compiler_params=pltpu.CompilerParams(use_tc_tiling_on_sc=False),
    )


def _sc_scatter_pair(edges, idx2_s, idx2_r, zeros_tab):
    return _make_sc_scatter()(edges, idx2_s, idx2_r, zeros_tab)


def _split_w1(w):
    return (w[0:64], w[64:128], w[128:192], w[192:256])


def _split_g1(w):
    return (w[0:64], w[64:128], w[128:192])


def _b2d(b):
    return b.reshape(1, -1)


def kernel(nodes, positions, box, edge_shifts, senders, receivers, params):
    p = params
    pad = EDGES_PAD - N_EDGES
    s32 = senders.astype(jnp.int32)
    r32 = receivers.astype(jnp.int32)
    idx_s = jnp.concatenate([s32, jnp.zeros((pad,), jnp.int32)]).reshape(
        IDX_ROWS, 128)
    idx_r = jnp.concatenate([r32, jnp.zeros((pad,), jnp.int32)]).reshape(
        IDX_ROWS, 128)
    shifts_pad = jnp.pad(edge_shifts, ((0, pad), (0, 13)))
    pos_pad = jnp.pad(positions, ((0, 0), (0, 13)))
    boxt = jnp.pad(box[0].T, ((0, 13), (0, 13)))
    zeros_tab = jnp.zeros((N_NODES, 64), F32)
    wee = jnp.pad(p['edge_emb']['W'], ((0, 2), (0, 0)))

    tab0, glob0 = _tc_embed(nodes, p['node_emb']['W'],
                            _b2d(p['node_emb']['b']), pos_pad,
                            p['glob_emb']['W'], _b2d(p['glob_emb']['b']))

    gs, gr = _sc_gather_pair(tab0, idx_s, idx_r)

    sp = p['steps'][0]
    edges_h, esum = _tc_edge_first(
        gs, gr, shifts_pad, boxt, wee, _b2d(p['edge_emb']['b']), glob0,
        _split_w1(sp['edge'][0]['W']), _b2d(sp['edge'][0]['b']),
        sp['edge'][1]['W'], _b2d(sp['edge'][1]['b']))

    nodes_h = tab0[:, :64]
    glob_h = glob0
    n_steps = len(p['steps'])
    for step_i in range(n_steps + 1):
        sp = p['steps'][step_i] if step_i < n_steps else p['readout']
        is_last = step_i == n_steps
        parts = _sc_scatter_pair(edges_h, idx_s, idx_r, zeros_tab)
        g2 = sp['global'][1]['W']
        bg2 = _b2d(sp['global'][1]['b'])
        if is_last:
            g2 = jnp.pad(g2, ((0, 0), (0, 64 - g2.shape[1])))
            bg2 = jnp.pad(bg2, ((0, 0), (0, 64 - bg2.shape[1])))
        nodes_h, glob_h = _tc_node(
            nodes_h, parts, esum, glob_h,
            _split_w1(sp['node'][0]['W']), _b2d(sp['node'][0]['b']),
            sp['node'][1]['W'], _b2d(sp['node'][1]['b']),
            _split_g1(sp['global'][0]['W']), _b2d(sp['global'][0]['b']),
            g2, bg2,
            out_scale=float(N_NODES) if is_last else 1.0)
        if is_last:
            break
        nsp = p['steps'][step_i + 1] if step_i + 1 < n_steps else p['readout']
        gs, gr = _sc_gather_pair(nodes_h, idx_s, idx_r)
        edges_h, esum = _tc_edge_block(
            edges_h, gs, gr, glob_h,
            _split_w1(nsp['edge'][0]['W']), _b2d(nsp['edge'][0]['b']),
            nsp['edge'][1]['W'], _b2d(nsp['edge'][1]['b']))

    return glob_h[:, :1]
